# Initial kernel scaffold; baseline (speedup 1.0000x reference)
import jax
import jax.numpy as jnp
from jax.experimental import pallas as pl


def kernel(x, emb_proto, emb_flags, emb_dir, W_len, b_len, W_iat, b_iat, W_fusion, b_fusion, ln_gamma, ln_beta):
    raise NotImplementedError("write your pallas kernel here")


# speedup vs baseline: 4.3162x; 4.3162x over previous
import jax
import jax.numpy as jnp
from jax import lax
from jax.experimental import pallas as pl

_RB = 1024
_KP = 384
_DM = 256


def _tables_body(ep_ref, ef_ref, ed_ref, wl_ref, bl_ref, wi_ref, bi_ref,
                 wf_ref, bf_ref, t_ref):
    wf = wf_ref[...]
    t_ref[0:256, :] = jnp.dot(ep_ref[...], wf[0:32, :],
                              preferred_element_type=jnp.float32)
    t_ref[256:320, :] = jnp.dot(ef_ref[...], wf[64:96, :],
                                preferred_element_type=jnp.float32)
    v_len = jnp.dot(wl_ref[...], wf[32:64, :],
                    preferred_element_type=jnp.float32)
    v_iat = jnp.dot(wi_ref[...], wf[96:128, :],
                    preferred_element_type=jnp.float32)
    td = jnp.dot(ed_ref[...], wf[128:136, :],
                 preferred_element_type=jnp.float32)
    c = (bf_ref[...]
         + jnp.dot(bl_ref[...], wf[32:64, :], preferred_element_type=jnp.float32)
         + jnp.dot(bi_ref[...], wf[96:128, :], preferred_element_type=jnp.float32)
         + td[0:1, :])
    d_vec = td[1:2, :] - td[0:1, :]
    tail = jnp.concatenate(
        [v_len, v_iat, d_vec, c, jnp.zeros((60, _DM), jnp.float32)], axis=0)
    t_ref[320:384, :] = tail


def _fwd_body(x_ref, t_ref, g_ref, b_ref, o_ref):
    xb = x_ref[...]
    pi = jnp.clip(xb[:, 0:1].astype(jnp.int32), 0, 255)
    fi = jnp.clip(xb[:, 2:3].astype(jnp.int32), 0, 63) + 256
    ln = xb[:, 1:2]
    it = xb[:, 3:4]
    dr = jnp.clip(xb[:, 4:5].astype(jnp.int32), 0, 1).astype(jnp.float32)
    iota = lax.broadcasted_iota(jnp.int32, (_RB, _KP), 1)
    onehot = jnp.where(jnp.logical_or(iota == pi, iota == fi),
                       jnp.float32(1.0), jnp.float32(0.0))
    t = t_ref[...]
    h = jnp.dot(onehot, t, preferred_element_type=jnp.float32)
    h = (h + ln * t[320:321, :] + it * t[321:322, :]
         + dr * t[322:323, :] + t[323:324, :])
    m = jnp.mean(h, axis=1, keepdims=True)
    d = h - m
    v = jnp.mean(d * d, axis=1, keepdims=True)
    o_ref[...] = d * lax.rsqrt(v + 1e-5) * g_ref[...] + b_ref[...]


def kernel(x, emb_proto, emb_flags, emb_dir, W_len, b_len, W_iat, b_iat,
           W_fusion, b_fusion, ln_gamma, ln_beta):
    B, L, _ = x.shape
    R = B * L
    x2 = x.reshape(R, 5)
    wfT = W_fusion.T
    t = pl.pallas_call(
        _tables_body,
        out_shape=jax.ShapeDtypeStruct((_KP, _DM), jnp.float32),
    )(emb_proto, emb_flags, emb_dir,
      W_len.reshape(1, 32), b_len.reshape(1, 32),
      W_iat.reshape(1, 32), b_iat.reshape(1, 32),
      wfT, b_fusion.reshape(1, _DM))

    out2 = pl.pallas_call(
        _fwd_body,
        grid=(R // _RB,),
        in_specs=[
            pl.BlockSpec((_RB, 5), lambda i: (i, 0)),
            pl.BlockSpec((_KP, _DM), lambda i: (0, 0)),
            pl.BlockSpec((1, _DM), lambda i: (0, 0)),
            pl.BlockSpec((1, _DM), lambda i: (0, 0)),
        ],
        out_specs=pl.BlockSpec((_RB, _DM), lambda i: (i, 0)),
        out_shape=jax.ShapeDtypeStruct((R, _DM), jnp.float32),
    )(x2, t, ln_gamma.reshape(1, _DM), ln_beta.reshape(1, _DM))
    return out2.reshape(B, L, _DM)


# speedup vs baseline: 6.0216x; 1.3951x over previous
import jax
import jax.numpy as jnp
from jax import lax
from jax.experimental import pallas as pl

_RB = 1600
_KP = 384
_DM = 256


def _tables_body(ep_ref, ef_ref, ed_ref, wl_ref, bl_ref, wi_ref, bi_ref,
                 wf_ref, bf_ref, t_ref):
    wf = wf_ref[...]
    t_ref[0:256, :] = jnp.dot(ep_ref[...], wf[0:32, :],
                              preferred_element_type=jnp.float32)
    t_ref[256:320, :] = jnp.dot(ef_ref[...], wf[64:96, :],
                                preferred_element_type=jnp.float32)
    v_len = jnp.dot(wl_ref[...], wf[32:64, :],
                    preferred_element_type=jnp.float32)
    v_iat = jnp.dot(wi_ref[...], wf[96:128, :],
                    preferred_element_type=jnp.float32)
    td = jnp.dot(ed_ref[...], wf[128:136, :],
                 preferred_element_type=jnp.float32)
    c = (bf_ref[...]
         + jnp.dot(bl_ref[...], wf[32:64, :], preferred_element_type=jnp.float32)
         + jnp.dot(bi_ref[...], wf[96:128, :], preferred_element_type=jnp.float32)
         + td[0:1, :])
    d_vec = td[1:2, :] - td[0:1, :]
    tail = jnp.concatenate(
        [v_len, v_iat, d_vec, c, jnp.zeros((60, _DM), jnp.float32)], axis=0)
    t_ref[320:384, :] = tail


def _fwd_body(x_ref, t_ref, g_ref, b_ref, o_ref):
    xb = x_ref[...]
    bb, ll, _ = o_ref.shape
    pi = jnp.clip(xb[:, 0:1].astype(jnp.int32), 0, 255)
    fi = jnp.clip(xb[:, 2:3].astype(jnp.int32), 0, 63) + 256
    ln = xb[:, 1:2]
    it = xb[:, 3:4]
    dr = jnp.clip(xb[:, 4:5].astype(jnp.int32), 0, 1).astype(jnp.float32)
    iota = lax.broadcasted_iota(jnp.int32, (_RB, _KP), 1)
    onehot = jnp.where(jnp.logical_or(iota == pi, iota == fi),
                       jnp.float32(1.0), jnp.float32(0.0))
    t = t_ref[...]
    h = jnp.dot(onehot, t, preferred_element_type=jnp.float32)
    h = (h + ln * t[320:321, :] + it * t[321:322, :]
         + dr * t[322:323, :] + t[323:324, :])
    m = jnp.mean(h, axis=1, keepdims=True)
    d = h - m
    v = jnp.mean(d * d, axis=1, keepdims=True)
    out = d * lax.rsqrt(v + 1e-5) * g_ref[...] + b_ref[...]
    o_ref[...] = out.reshape(bb, ll, _DM)


def kernel(x, emb_proto, emb_flags, emb_dir, W_len, b_len, W_iat, b_iat,
           W_fusion, b_fusion, ln_gamma, ln_beta):
    B, L, _ = x.shape
    R = B * L
    x2 = x.reshape(R, 5)
    wfT = W_fusion.T
    t = pl.pallas_call(
        _tables_body,
        out_shape=jax.ShapeDtypeStruct((_KP, _DM), jnp.float32),
    )(emb_proto, emb_flags, emb_dir,
      W_len.reshape(1, 32), b_len.reshape(1, 32),
      W_iat.reshape(1, 32), b_iat.reshape(1, 32),
      wfT, b_fusion.reshape(1, _DM))

    bb = _RB // L
    out3 = pl.pallas_call(
        _fwd_body,
        grid=(R // _RB,),
        in_specs=[
            pl.BlockSpec((_RB, 5), lambda i: (i, 0)),
            pl.BlockSpec((_KP, _DM), lambda i: (0, 0)),
            pl.BlockSpec((1, _DM), lambda i: (0, 0)),
            pl.BlockSpec((1, _DM), lambda i: (0, 0)),
        ],
        out_specs=pl.BlockSpec((bb, L, _DM), lambda i: (i, 0, 0)),
        out_shape=jax.ShapeDtypeStruct((B, L, _DM), jnp.float32),
    )(x2, t, ln_gamma.reshape(1, _DM), ln_beta.reshape(1, _DM))
    return out3
